# baseline (speedup 1.0000x reference)
import functools

import jax
import jax.numpy as jnp
from jax import lax
from jax.experimental import pallas as pl
from jax.experimental.pallas import tpu as pltpu
from jax.experimental.pallas import tpu_sc as plsc

N = 10000
E = 320000
D = 128
H = 128
T = 12
OUT = 128
NHEADS = 2
DH = H // NHEADS

NC = 2
NS = 16
NW = NC * NS
L = 16

CHK = 64
EW_PAD = ((E // NW) + 2 * CHK - 1) // (2 * CHK) * (2 * CHK)
NCHK = EW_PAD // CHK
EP = EW_PAD * NW
N_PAD = 10240
ROWS_PER_SUB = N_PAD // NS


def _sc_agg_body(xp_hbm, src_hbm, dst_hbm, nrm_hbm, zeros_hbm, out_hbm,
                 sbuf, dbuf, nbuf, rows, acc, *sems):
    sems_s = sems[0:2]
    sems_d = sems[2:4]
    semg = sems[4:6]
    semsc = sems[6:8]
    c = lax.axis_index("c")
    s = lax.axis_index("s")
    w = c * NS + s
    r0 = s * ROWS_PER_SUB
    pltpu.sync_copy(zeros_hbm.at[pl.ds(r0, ROWS_PER_SUB)],
                    acc.at[pl.ds(r0, ROWS_PER_SUB)])
    plsc.subcore_barrier()

    def issue_src(ch, b):
        pltpu.async_copy(src_hbm.at[w, ch], sbuf.at[b], sems_s[b])
        pltpu.async_copy(nrm_hbm.at[w, ch], nbuf.at[b], sems_s[b])

    def wait_src(ch, b):
        pltpu.make_async_copy(src_hbm.at[w, ch], sbuf.at[b],
                              sems_s[b]).wait()
        pltpu.make_async_copy(nrm_hbm.at[w, ch], nbuf.at[b],
                              sems_s[b]).wait()

    def issue_dst(ch, b):
        pltpu.async_copy(dst_hbm.at[w, ch], dbuf.at[b], sems_d[b])

    def wait_dst(ch, b):
        pltpu.make_async_copy(dst_hbm.at[w, ch], dbuf.at[b],
                              sems_d[b]).wait()

    def issue_gather(b):
        pltpu.async_copy(xp_hbm.at[sbuf.at[b]], rows.at[b], semg[b])

    def wait_gather(b):
        pltpu.make_async_copy(xp_hbm.at[sbuf.at[b]], rows.at[b],
                              semg[b]).wait()

    def issue_scatter(b):
        pltpu.async_copy(rows.at[b], acc.at[dbuf.at[b]], semsc[b], add=True)

    def wait_scatter(b):
        pltpu.make_async_copy(rows.at[b], acc.at[dbuf.at[b]],
                              semsc[b]).wait()

    def scale(b):
        def grp(g, cr):
            for j in range(8):
                e = g * 8 + j
                bc = nbuf[b, e]
                for q in range(H // L):
                    rows[b, e, pl.ds(q * L, L)] = (
                        rows[b, e, pl.ds(q * L, L)] * bc)
            return cr
        lax.fori_loop(0, CHK // 8, grp, 0)

    z = jnp.int32(0)
    issue_src(z, 0)
    issue_src(z + 1, 1)
    issue_dst(z, 0)
    wait_src(z, 0)
    issue_gather(0)

    def loop_body(m, carry):
        for b in range(2):
            ch = 2 * m + b
            nb = 1 - b
            wait_gather(b)

            @pl.when(ch > 0)
            def _():
                wait_scatter(nb)

            @pl.when(ch + 1 < NCHK)
            def _():
                issue_dst(ch + 1, nb)
                wait_src(ch + 1, nb)
                issue_gather(nb)
            scale(b)
            wait_dst(ch, b)
            issue_scatter(b)

            @pl.when(ch + 2 < NCHK)
            def _():
                issue_src(ch + 2, b)
        return carry

    lax.fori_loop(0, NCHK // 2, loop_body, 0)
    wait_scatter(1)
    plsc.subcore_barrier()
    pltpu.sync_copy(acc.at[pl.ds(r0, ROWS_PER_SUB)],
                    out_hbm.at[c, pl.ds(r0, ROWS_PER_SUB)])


@functools.cache
def _sc_agg_kernel():
    mesh = plsc.VectorSubcoreMesh(
        core_axis_name="c", subcore_axis_name="s",
        num_cores=NC, num_subcores=NS)
    return pl.kernel(
        _sc_agg_body,
        out_type=jax.ShapeDtypeStruct((NC, N_PAD, H), jnp.float32),
        mesh=mesh,
        scratch_types=[
            pltpu.VMEM((2, CHK), jnp.int32),
            pltpu.VMEM((2, CHK), jnp.int32),
            pltpu.VMEM((2, CHK, L), jnp.float32),
            pltpu.VMEM((2, CHK, H), jnp.float32),
            pltpu.VMEM_SHARED((N_PAD, H), jnp.float32),
        ] + [pltpu.SemaphoreType.DMA] * 8,
    )


def _gru_math(xt, p0, p1, xp, ivd, h, gb,
              wux, wug, wuh, bu, wrx, wrg, wrh, br, wcx, wcg, wch, bc):
    g = jax.nn.sigmoid(p0 + p1 + xp * ivd + gb)
    u = jax.nn.sigmoid(xt @ wux + g @ wug + h @ wuh + bu)
    r = jax.nn.sigmoid(xt @ wrx + g @ wrg + h @ wrh + br)
    cc = jnp.tanh(xt @ wcx + g @ wcg + (r * h) @ wch + bc)
    return u * h + (1.0 - u) * cc


def _mha_math(hfb, inw, inb, outw, outb, ow, ob):
    qs, ks, vs = [], [], []
    for t in range(T):
        qkv = hfb[t] @ inw + inb
        qs.append(qkv[:, :H])
        ks.append(qkv[:, H:2 * H])
        vs.append(qkv[:, 2 * H:])
    scale = 1.0 / (DH ** 0.5)
    head_outs = []
    for hd in range(NHEADS):
        lo, hi = hd * DH, (hd + 1) * DH
        acc = None
        for i in range(T):
            qi = qs[i][:, lo:hi]
            sc = jnp.concatenate(
                [jnp.sum(qi * ks[j][:, lo:hi], axis=1, keepdims=True) * scale
                 for j in range(T)], axis=1)
            a = jax.nn.softmax(sc, axis=1)
            oi = None
            for j in range(T):
                term = a[:, j:j + 1] * vs[j][:, lo:hi]
                oi = term if oi is None else oi + term
            acc = oi if acc is None else acc + oi
        head_outs.append(acc * (1.0 / T))
    m = jnp.concatenate(head_outs, axis=1)
    return (m @ outw + outb) @ ow + ob


BN_XP = 1024
BN_GRU = 1024
BN_MHA = 256


def _xp_all_call(xT, w0T):
    nb = pl.cdiv(N, BN_XP)

    def body(x_ref, w_ref, o_ref):
        o_ref[0] = jnp.dot(x_ref[0], w_ref[...],
                           preferred_element_type=jnp.float32)

    return pl.pallas_call(
        body,
        grid=(T, nb),
        in_specs=[
            pl.BlockSpec((1, BN_XP, D), lambda t, i: (t, i, 0)),
            pl.BlockSpec((D, H), lambda t, i: (0, 0)),
        ],
        out_specs=pl.BlockSpec((1, BN_XP, H), lambda t, i: (t, i, 0)),
        out_shape=jax.ShapeDtypeStruct((T, N, H), jnp.float32),
    )(xT, w0T)


def _row_blk(r):
    return pl.BlockSpec((BN_GRU, r), lambda i: (i, 0))


def _full2(a, b):
    return pl.BlockSpec((a, b), lambda i: (0, 0))


def _full1(a):
    return pl.BlockSpec((a,), lambda i: (0,))


def _gru_call(xt, part, xp, ivd, h, gb, ws, compute_xp, w1T=None):
    nb = pl.cdiv(N, BN_GRU)

    if compute_xp:
        def body(xt_r, p0_r, p1_r, xp_r, ivd_r, h_r, gb_r,
                 wux, wug, wuh, bu, wrx, wrg, wrh, br, wcx, wcg, wch, bc,
                 w1_r, hn_r, xpn_r):
            hn = _gru_math(xt_r[...], p0_r[0], p1_r[0], xp_r[...],
                           ivd_r[...], h_r[...], gb_r[...],
                           wux[...], wug[...], wuh[...], bu[...],
                           wrx[...], wrg[...], wrh[...], br[...],
                           wcx[...], wcg[...], wch[...], bc[...])
            hn_r[...] = hn
            xpn_r[...] = jnp.dot(hn, w1_r[...],
                                 preferred_element_type=jnp.float32)
        out_shape = (jax.ShapeDtypeStruct((N, H), jnp.float32),
                     jax.ShapeDtypeStruct((N, H), jnp.float32))
        out_specs = (_row_blk(H), _row_blk(H))
        extra_in = [_full2(H, H)]
        args = (xt, part, part, xp, ivd, h, gb) + ws + (w1T,)
    else:
        def body(xt_r, p0_r, p1_r, xp_r, ivd_r, h_r, gb_r,
                 wux, wug, wuh, bu, wrx, wrg, wrh, br, wcx, wcg, wch, bc,
                 hn_r):
            hn_r[...] = _gru_math(xt_r[...], p0_r[0], p1_r[0], xp_r[...],
                                  ivd_r[...], h_r[...], gb_r[...],
                                  wux[...], wug[...], wuh[...], bu[...],
                                  wrx[...], wrg[...], wrh[...], br[...],
                                  wcx[...], wcg[...], wch[...], bc[...])
        out_shape = jax.ShapeDtypeStruct((N, H), jnp.float32)
        out_specs = _row_blk(H)
        extra_in = []
        args = (xt, part, part, xp, ivd, h, gb) + ws

    p0_spec = pl.BlockSpec((1, BN_GRU, H), lambda i: (0, i, 0))
    p1_spec = pl.BlockSpec((1, BN_GRU, H), lambda i: (1, i, 0))
    w_specs = [
        _full2(D, H), _full2(H, H), _full2(H, H), _full1(H),
        _full2(D, H), _full2(H, H), _full2(H, H), _full1(H),
        _full2(D, H), _full2(H, H), _full2(H, H), _full1(H),
    ]
    return pl.pallas_call(
        body,
        grid=(nb,),
        in_specs=[_row_blk(D), p0_spec, p1_spec, _row_blk(H),
                  pl.BlockSpec((BN_GRU, 1), lambda i: (i, 0)), _row_blk(H),
                  _full1(H)] + w_specs + extra_in,
        out_specs=out_specs,
        out_shape=out_shape,
    )(*args)


def _mha_call(hf, inwT, inb, outwT, outb, owT, ob):
    nb = pl.cdiv(N, BN_MHA)

    def body(hf_r, inw_r, inb_r, outw_r, outb_r, ow_r, ob_r, o_r):
        o_r[...] = _mha_math(hf_r[...], inw_r[...], inb_r[...],
                             outw_r[...], outb_r[...], ow_r[...], ob_r[...])

    return pl.pallas_call(
        body,
        grid=(nb,),
        in_specs=[
            pl.BlockSpec((T, BN_MHA, H), lambda i: (0, i, 0)),
            _full2(H, 3 * H), _full1(3 * H),
            _full2(H, H), _full1(H),
            _full2(H, OUT), _full1(OUT),
        ],
        out_specs=pl.BlockSpec((BN_MHA, OUT), lambda i: (i, 0)),
        out_shape=jax.ShapeDtypeStruct((N, OUT), jnp.float32),
    )(hf, inwT, inb, outwT, outb, owT, ob)


def kernel(x, edge_index, edge_attr, gcn_w0, gcn_b0, wu_w0, wu_b0, wr_w0,
           wr_b0, wc_w0, wc_b0, gcn_w1, gcn_b1, wu_w1, wu_b1, wr_w1, wr_b1,
           wc_w1, wc_b1, attn_in_w, attn_in_b, attn_out_w, attn_out_b,
           out_w, out_b):
    f32 = jnp.float32
    ew = edge_attr[:, -1]
    src = edge_index[0]
    dst = edge_index[1]

    deg = jnp.zeros((N,), f32).at[dst].add(ew) + 1.0
    dinv = lax.rsqrt(deg)
    norm = dinv[src] * ew * dinv[dst]
    ivd = (1.0 / deg)[:, None]

    order = jnp.argsort(src)
    src = src[order]
    dst = dst[order]
    norm = norm[order]

    pad = EP - E
    srcp = jnp.concatenate([src, jnp.zeros((pad,), src.dtype)]).reshape(
        NW, NCHK, CHK)
    dstp = jnp.concatenate([dst, jnp.zeros((pad,), dst.dtype)]).reshape(
        NW, NCHK, CHK)
    nrmp = jnp.broadcast_to(
        jnp.concatenate([norm, jnp.zeros((pad,), f32)])[:, None],
        (EP, L)).reshape(NW, NCHK, CHK, L)
    zeros_nh = jnp.zeros((N_PAD, H), f32)

    def agg(xp):
        return _sc_agg_kernel()(xp, srcp, dstp, nrmp, zeros_nh)

    def split3(w, d0):
        return (w[:, :d0].T, w[:, d0:d0 + H].T, w[:, d0 + H:].T)

    ws0 = (split3(wu_w0, D) + (wu_b0,) + split3(wr_w0, D) + (wr_b0,)
           + split3(wc_w0, D) + (wc_b0,))
    ws1 = (split3(wu_w1, H) + (wu_b1,) + split3(wr_w1, H) + (wr_b1,)
           + split3(wc_w1, H) + (wc_b1,))

    xT = jnp.transpose(x, (2, 0, 1))
    xp0_all = _xp_all_call(xT, gcn_w0.T)

    h0 = jnp.zeros((N, H), f32)
    h1 = jnp.zeros((N, H), f32)
    hs = []
    for t in range(T):
        xt = xT[t]
        xp0 = xp0_all[t]
        part0 = agg(xp0)
        h0, xp1 = _gru_call(xt, part0, xp0, ivd, h0, gcn_b0, ws0,
                            compute_xp=True, w1T=gcn_w1.T)
        part1 = agg(xp1)
        h1 = _gru_call(h0, part1, xp1, ivd, h1, gcn_b1, ws1,
                       compute_xp=False)
        hs.append(h1)

    hf = jnp.stack(hs, axis=0)
    return _mha_call(hf, attn_in_w.T, attn_in_b, attn_out_w.T, attn_out_b,
                     out_w.T, out_b)


# speedup vs baseline: 1.2617x; 1.2617x over previous
import functools

import jax
import jax.numpy as jnp
from jax import lax
from jax.experimental import pallas as pl
from jax.experimental.pallas import tpu as pltpu
from jax.experimental.pallas import tpu_sc as plsc

N = 10000
E = 320000
D = 128
H = 128
T = 12
OUT = 128
NHEADS = 2
DH = H // NHEADS

NC = 2
NS = 16
NW = NC * NS
L = 16

CHK = 64
EW_PAD = ((E // NW) + 2 * CHK - 1) // (2 * CHK) * (2 * CHK)
NCHK = EW_PAD // CHK
EP = EW_PAD * NW
N_PAD = 10240
ROWS_PER_SUB = N_PAD // NS


def _sc_agg_body(xp_hbm, src_hbm, dst_hbm, nrm_hbm, zeros_hbm, out_hbm,
                 sbuf, dbuf, nbuf, rows, acc, *sems):
    sems_s = sems[0:2]
    sems_d = sems[2:4]
    semg = sems[4:6]
    semsc = sems[6:8]
    c = lax.axis_index("c")
    s = lax.axis_index("s")
    w = c * NS + s
    r0 = s * ROWS_PER_SUB
    pltpu.sync_copy(zeros_hbm.at[pl.ds(r0, ROWS_PER_SUB)],
                    acc.at[pl.ds(r0, ROWS_PER_SUB)])
    plsc.subcore_barrier()

    def issue_src(ch, b):
        pltpu.async_copy(src_hbm.at[w, ch], sbuf.at[b], sems_s[b])
        pltpu.async_copy(nrm_hbm.at[w, ch], nbuf.at[b], sems_s[b])

    def wait_src(ch, b):
        pltpu.make_async_copy(src_hbm.at[w, ch], sbuf.at[b],
                              sems_s[b]).wait()
        pltpu.make_async_copy(nrm_hbm.at[w, ch], nbuf.at[b],
                              sems_s[b]).wait()

    def issue_dst(ch, b):
        pltpu.async_copy(dst_hbm.at[w, ch], dbuf.at[b], sems_d[b])

    def wait_dst(ch, b):
        pltpu.make_async_copy(dst_hbm.at[w, ch], dbuf.at[b],
                              sems_d[b]).wait()

    def issue_gather(b):
        pltpu.async_copy(xp_hbm.at[sbuf.at[b]], rows.at[b], semg[b])

    def wait_gather(b):
        pltpu.make_async_copy(xp_hbm.at[sbuf.at[b]], rows.at[b],
                              semg[b]).wait()

    def issue_scatter(b):
        pltpu.async_copy(rows.at[b], acc.at[dbuf.at[b]], semsc[b], add=True)

    def wait_scatter(b):
        pltpu.make_async_copy(rows.at[b], acc.at[dbuf.at[b]],
                              semsc[b]).wait()

    def scale(b):
        def grp(g, cr):
            for j in range(8):
                e = g * 8 + j
                bc = nbuf[b, e]
                for q in range(H // L):
                    rows[b, e, pl.ds(q * L, L)] = (
                        rows[b, e, pl.ds(q * L, L)] * bc)
            return cr
        lax.fori_loop(0, CHK // 8, grp, 0)

    z = jnp.int32(0)
    issue_src(z, 0)
    issue_src(z + 1, 1)
    issue_dst(z, 0)
    wait_src(z, 0)
    issue_gather(0)

    def loop_body(m, carry):
        for b in range(2):
            ch = 2 * m + b
            nb = 1 - b
            wait_gather(b)

            @pl.when(ch > 0)
            def _():
                wait_scatter(nb)

            @pl.when(ch + 1 < NCHK)
            def _():
                issue_dst(ch + 1, nb)
                wait_src(ch + 1, nb)
                issue_gather(nb)
            scale(b)
            wait_dst(ch, b)
            issue_scatter(b)

            @pl.when(ch + 2 < NCHK)
            def _():
                issue_src(ch + 2, b)
        return carry

    lax.fori_loop(0, NCHK // 2, loop_body, 0)
    wait_scatter(1)
    plsc.subcore_barrier()
    pltpu.sync_copy(acc.at[pl.ds(r0, ROWS_PER_SUB)],
                    out_hbm.at[c, pl.ds(r0, ROWS_PER_SUB)])


@functools.cache
def _sc_agg_kernel():
    mesh = plsc.VectorSubcoreMesh(
        core_axis_name="c", subcore_axis_name="s",
        num_cores=NC, num_subcores=NS)
    return pl.kernel(
        _sc_agg_body,
        out_type=jax.ShapeDtypeStruct((NC, N_PAD, H), jnp.float32),
        mesh=mesh,
        scratch_types=[
            pltpu.VMEM((2, CHK), jnp.int32),
            pltpu.VMEM((2, CHK), jnp.int32),
            pltpu.VMEM((2, CHK, L), jnp.float32),
            pltpu.VMEM((2, CHK, H), jnp.float32),
            pltpu.VMEM_SHARED((N_PAD, H), jnp.float32),
        ] + [pltpu.SemaphoreType.DMA] * 8,
    )


def _gru_math(xt, p0, p1, xp, ivd, h, gb,
              wux, wug, wuh, bu, wrx, wrg, wrh, br, wcx, wcg, wch, bc):
    g = jax.nn.sigmoid(p0 + p1 + xp * ivd + gb)
    u = jax.nn.sigmoid(xt @ wux + g @ wug + h @ wuh + bu)
    r = jax.nn.sigmoid(xt @ wrx + g @ wrg + h @ wrh + br)
    cc = jnp.tanh(xt @ wcx + g @ wcg + (r * h) @ wch + bc)
    return u * h + (1.0 - u) * cc


def _mha_math(hfb, inw, inb, outw, outb, ow, ob):
    qs, ks, vs = [], [], []
    for t in range(T):
        qkv = hfb[t] @ inw + inb
        qs.append(qkv[:, :H])
        ks.append(qkv[:, H:2 * H])
        vs.append(qkv[:, 2 * H:])
    scale = 1.0 / (DH ** 0.5)
    head_outs = []
    for hd in range(NHEADS):
        lo, hi = hd * DH, (hd + 1) * DH
        acc = None
        for i in range(T):
            qi = qs[i][:, lo:hi]
            sc = jnp.concatenate(
                [jnp.sum(qi * ks[j][:, lo:hi], axis=1, keepdims=True) * scale
                 for j in range(T)], axis=1)
            a = jax.nn.softmax(sc, axis=1)
            oi = None
            for j in range(T):
                term = a[:, j:j + 1] * vs[j][:, lo:hi]
                oi = term if oi is None else oi + term
            acc = oi if acc is None else acc + oi
        head_outs.append(acc * (1.0 / T))
    m = jnp.concatenate(head_outs, axis=1)
    return (m @ outw + outb) @ ow + ob


BN_XP = 1024
BN_GRU = 1024
BN_MHA = 512


def _xp_all_call(xT, w0T):
    nb = pl.cdiv(N, BN_XP)

    def body(x_ref, w_ref, o_ref):
        o_ref[0] = jnp.dot(x_ref[0], w_ref[...],
                           preferred_element_type=jnp.float32)

    return pl.pallas_call(
        body,
        grid=(T, nb),
        in_specs=[
            pl.BlockSpec((1, BN_XP, D), lambda t, i: (t, i, 0)),
            pl.BlockSpec((D, H), lambda t, i: (0, 0)),
        ],
        out_specs=pl.BlockSpec((1, BN_XP, H), lambda t, i: (t, i, 0)),
        out_shape=jax.ShapeDtypeStruct((T, N, H), jnp.float32),
    )(xT, w0T)


def _row_blk(r):
    return pl.BlockSpec((BN_GRU, r), lambda i: (i, 0))


def _full2(a, b):
    return pl.BlockSpec((a, b), lambda i: (0, 0))


def _full1(a):
    return pl.BlockSpec((a,), lambda i: (0,))


def _gru_call(xt, part, xp, ivd, h, gb, ws, compute_xp, w1T=None):
    nb = pl.cdiv(N, BN_GRU)

    if compute_xp:
        def body(xt_r, p0_r, p1_r, xp_r, ivd_r, h_r, gb_r,
                 wux, wug, wuh, bu, wrx, wrg, wrh, br, wcx, wcg, wch, bc,
                 w1_r, hn_r, xpn_r):
            hn = _gru_math(xt_r[...], p0_r[0], p1_r[0], xp_r[...],
                           ivd_r[...], h_r[...], gb_r[...],
                           wux[...], wug[...], wuh[...], bu[...],
                           wrx[...], wrg[...], wrh[...], br[...],
                           wcx[...], wcg[...], wch[...], bc[...])
            hn_r[...] = hn
            xpn_r[...] = jnp.dot(hn, w1_r[...],
                                 preferred_element_type=jnp.float32)
        out_shape = (jax.ShapeDtypeStruct((N, H), jnp.float32),
                     jax.ShapeDtypeStruct((N, H), jnp.float32))
        out_specs = (_row_blk(H), _row_blk(H))
        extra_in = [_full2(H, H)]
        args = (xt, part, part, xp, ivd, h, gb) + ws + (w1T,)
    else:
        def body(xt_r, p0_r, p1_r, xp_r, ivd_r, h_r, gb_r,
                 wux, wug, wuh, bu, wrx, wrg, wrh, br, wcx, wcg, wch, bc,
                 hn_r):
            hn_r[...] = _gru_math(xt_r[...], p0_r[0], p1_r[0], xp_r[...],
                                  ivd_r[...], h_r[...], gb_r[...],
                                  wux[...], wug[...], wuh[...], bu[...],
                                  wrx[...], wrg[...], wrh[...], br[...],
                                  wcx[...], wcg[...], wch[...], bc[...])
        out_shape = jax.ShapeDtypeStruct((N, H), jnp.float32)
        out_specs = _row_blk(H)
        extra_in = []
        args = (xt, part, part, xp, ivd, h, gb) + ws

    p0_spec = pl.BlockSpec((1, BN_GRU, H), lambda i: (0, i, 0))
    p1_spec = pl.BlockSpec((1, BN_GRU, H), lambda i: (1, i, 0))
    w_specs = [
        _full2(D, H), _full2(H, H), _full2(H, H), _full1(H),
        _full2(D, H), _full2(H, H), _full2(H, H), _full1(H),
        _full2(D, H), _full2(H, H), _full2(H, H), _full1(H),
    ]
    return pl.pallas_call(
        body,
        grid=(nb,),
        in_specs=[_row_blk(D), p0_spec, p1_spec, _row_blk(H),
                  pl.BlockSpec((BN_GRU, 1), lambda i: (i, 0)), _row_blk(H),
                  _full1(H)] + w_specs + extra_in,
        out_specs=out_specs,
        out_shape=out_shape,
    )(*args)


def _mha_call(hf, inwT, inb, outwT, outb, owT, ob):
    nb = pl.cdiv(N, BN_MHA)

    def body(hf_r, inw_r, inb_r, outw_r, outb_r, ow_r, ob_r, o_r):
        o_r[...] = _mha_math(hf_r[...], inw_r[...], inb_r[...],
                             outw_r[...], outb_r[...], ow_r[...], ob_r[...])

    return pl.pallas_call(
        body,
        grid=(nb,),
        in_specs=[
            pl.BlockSpec((T, BN_MHA, H), lambda i: (0, i, 0)),
            _full2(H, 3 * H), _full1(3 * H),
            _full2(H, H), _full1(H),
            _full2(H, OUT), _full1(OUT),
        ],
        out_specs=pl.BlockSpec((BN_MHA, OUT), lambda i: (i, 0)),
        out_shape=jax.ShapeDtypeStruct((N, OUT), jnp.float32),
    )(hf, inwT, inb, outwT, outb, owT, ob)


def kernel(x, edge_index, edge_attr, gcn_w0, gcn_b0, wu_w0, wu_b0, wr_w0,
           wr_b0, wc_w0, wc_b0, gcn_w1, gcn_b1, wu_w1, wu_b1, wr_w1, wr_b1,
           wc_w1, wc_b1, attn_in_w, attn_in_b, attn_out_w, attn_out_b,
           out_w, out_b):
    f32 = jnp.float32
    ew = edge_attr[:, -1]
    src = edge_index[0]
    dst = edge_index[1]

    deg = jnp.zeros((N,), f32).at[dst].add(ew) + 1.0
    dinv = lax.rsqrt(deg)
    norm = dinv[src] * ew * dinv[dst]
    ivd = (1.0 / deg)[:, None]

    pad = EP - E
    srcp = jnp.concatenate([src, jnp.zeros((pad,), src.dtype)]).reshape(
        NW, NCHK, CHK)
    dstp = jnp.concatenate([dst, jnp.zeros((pad,), dst.dtype)]).reshape(
        NW, NCHK, CHK)
    nrmp = jnp.broadcast_to(
        jnp.concatenate([norm, jnp.zeros((pad,), f32)])[:, None],
        (EP, L)).reshape(NW, NCHK, CHK, L)
    zeros_nh = jnp.zeros((N_PAD, H), f32)

    def agg(xp):
        return _sc_agg_kernel()(xp, srcp, dstp, nrmp, zeros_nh)

    def split3(w, d0):
        return (w[:, :d0].T, w[:, d0:d0 + H].T, w[:, d0 + H:].T)

    ws0 = (split3(wu_w0, D) + (wu_b0,) + split3(wr_w0, D) + (wr_b0,)
           + split3(wc_w0, D) + (wc_b0,))
    ws1 = (split3(wu_w1, H) + (wu_b1,) + split3(wr_w1, H) + (wr_b1,)
           + split3(wc_w1, H) + (wc_b1,))

    xT = jnp.transpose(x, (2, 0, 1))
    xp0_all = _xp_all_call(xT, gcn_w0.T)

    h0 = jnp.zeros((N, H), f32)
    h1 = jnp.zeros((N, H), f32)
    hs = []
    for t in range(T):
        xt = xT[t]
        xp0 = xp0_all[t]
        part0 = agg(xp0)
        h0, xp1 = _gru_call(xt, part0, xp0, ivd, h0, gcn_b0, ws0,
                            compute_xp=True, w1T=gcn_w1.T)
        part1 = agg(xp1)
        h1 = _gru_call(h0, part1, xp1, ivd, h1, gcn_b1, ws1,
                       compute_xp=False)
        hs.append(h1)

    hf = jnp.stack(hs, axis=0)
    return _mha_call(hf, attn_in_w.T, attn_in_b, attn_out_w.T, attn_out_b,
                     out_w.T, out_b)
